# baseline (speedup 1.0000x reference)
import functools

import jax
import jax.numpy as jnp
from jax import lax
from jax.experimental import pallas as pl
from jax.experimental.pallas import tpu as pltpu
from jax.experimental.pallas import tpu_sc as plsc

BATCH = 16384
SEQ = 50
DIM = 65
PAD_DIM = 128
PAD_SEQ = 56
NUM_WORKERS = 32
PAIRS = BATCH // 2
PAIRS_PER_W = PAIRS // NUM_WORKERS
SLAB_ROWS = BATCH * PAD_SEQ

PAD_BLK = 2000
FMT_B = 8


def _pad_kernel(w_ref, o_ref):
    o_ref[:, :DIM] = w_ref[...]


def _tc_pad(weight):
    return pl.pallas_call(
        _pad_kernel,
        grid=(weight.shape[0] // PAD_BLK,),
        in_specs=[pl.BlockSpec((PAD_BLK, DIM), lambda i: (i, 0))],
        out_specs=pl.BlockSpec((PAD_BLK, PAD_DIM), lambda i: (i, 0)),
        out_shape=jax.ShapeDtypeStruct((weight.shape[0], PAD_DIM), jnp.float32),
    )(weight)


def _gather_kernel(idx_hbm, table_hbm, out_hbm, idx_v, rows0, rows1, sem):
    wid = lax.axis_index("s") * 2 + lax.axis_index("c")
    g0 = wid * PAIRS_PER_W
    pltpu.sync_copy(idx_hbm.at[pl.ds(g0, PAIRS_PER_W)], idx_v)
    bufs = (rows0, rows1)

    def gather(g, buf):
        pltpu.async_copy(
            table_hbm.at[idx_v.at[g].at[pl.ds(0, 2 * SEQ)]],
            buf.at[pl.ds(0, 2 * SEQ)], sem
        )

    def drain(buf):
        pltpu.make_async_copy(table_hbm.at[idx_v.at[0].at[pl.ds(0, 2 * SEQ)]],
                              buf.at[pl.ds(0, 2 * SEQ)], sem).wait()

    gather(0, rows0)

    def body(g2, _):
        for k in range(2):
            g = 2 * g2 + k
            buf = bufs[k]

            @pl.when(g + 1 < PAIRS_PER_W)
            def _():
                gather(g + 1, bufs[1 - k])

            drain(buf)
            base = (g0 + g) * 2 * PAD_SEQ
            pltpu.sync_copy(buf.at[pl.ds(0, PAD_SEQ)],
                            out_hbm.at[pl.ds(base, PAD_SEQ)])
            pltpu.sync_copy(buf.at[pl.ds(SEQ, PAD_SEQ)],
                            out_hbm.at[pl.ds(base + PAD_SEQ, PAD_SEQ)])
        return 0

    lax.fori_loop(0, PAIRS_PER_W // 2, body, 0)


def _sc_gather(idx, table):
    mesh = plsc.VectorSubcoreMesh(core_axis_name="c", subcore_axis_name="s")
    k = functools.partial(
        pl.kernel,
        mesh=mesh,
        out_type=jax.ShapeDtypeStruct((SLAB_ROWS, PAD_DIM), jnp.float32),
        scratch_types=[
            pltpu.VMEM((PAIRS_PER_W, PAD_DIM), jnp.int32),
            pltpu.VMEM((2 * PAD_SEQ, PAD_DIM), jnp.float32),
            pltpu.VMEM((2 * PAD_SEQ, PAD_DIM), jnp.float32),
            pltpu.SemaphoreType.DMA,
        ],
    )(_gather_kernel)
    return k(idx, table)


def _fmt_kernel(slab_ref, o_ref):
    for k in range(FMT_B):
        o_ref[k] = slab_ref[pl.ds(PAD_SEQ * k, SEQ), :DIM]


def _tc_format(slab):
    return pl.pallas_call(
        _fmt_kernel,
        grid=(BATCH // FMT_B,),
        in_specs=[pl.BlockSpec((FMT_B * PAD_SEQ, PAD_DIM), lambda i: (i, 0))],
        out_specs=pl.BlockSpec((FMT_B, SEQ, DIM), lambda i: (i, 0, 0)),
        out_shape=jax.ShapeDtypeStruct((BATCH, SEQ, DIM), jnp.float32),
    )(slab)


def kernel(indices, weight):
    table = _tc_pad(weight.astype(jnp.float32))
    idx = jnp.pad(indices.reshape(PAIRS, 2 * SEQ).astype(jnp.int32),
                  ((0, 0), (0, PAD_DIM - 2 * SEQ)))
    slab = _sc_gather(idx, table)
    return _tc_format(slab)


# speedup vs baseline: 1.5031x; 1.5031x over previous
import functools

import jax
import jax.numpy as jnp
from jax import lax
from jax.experimental import pallas as pl
from jax.experimental.pallas import tpu as pltpu
from jax.experimental.pallas import tpu_sc as plsc

BATCH = 16384
SEQ = 50
DIM = 65
PAD_DIM = 128
NUM_ROWS = BATCH * SEQ
NUM_WORKERS = 32
QUADS = NUM_ROWS // 200
Q_PER_W = QUADS // NUM_WORKERS
GROUP = 200
HALF = 100

PAD_BLK = 2000


def _pad_kernel(w_ref, o_ref):
    o_ref[:, :DIM] = w_ref[...]


def _tc_pad(weight):
    return pl.pallas_call(
        _pad_kernel,
        grid=(weight.shape[0] // PAD_BLK,),
        in_specs=[pl.BlockSpec((PAD_BLK, DIM), lambda i: (i, 0))],
        out_specs=pl.BlockSpec((PAD_BLK, PAD_DIM), lambda i: (i, 0)),
        out_shape=jax.ShapeDtypeStruct((weight.shape[0], PAD_DIM), jnp.float32),
    )(weight)


def _gather_kernel(idx_hbm, table_hbm, out_hbm, idx_v, rows0, rows1, sem):
    wid = lax.axis_index("s") * 2 + lax.axis_index("c")
    q0 = wid * Q_PER_W
    pltpu.sync_copy(idx_hbm.at[pl.ds(q0, Q_PER_W)], idx_v)
    bufs = (rows0, rows1)

    def gather(q, buf):
        pltpu.async_copy(table_hbm.at[idx_v.at[q].at[pl.ds(0, HALF)]],
                         buf.at[pl.ds(0, HALF)], sem)
        pltpu.async_copy(table_hbm.at[idx_v.at[q].at[pl.ds(PAD_DIM, HALF)]],
                         buf.at[pl.ds(HALF, HALF)], sem)

    def drain(buf):
        pltpu.make_async_copy(table_hbm.at[idx_v.at[0].at[pl.ds(0, HALF)]],
                              buf.at[pl.ds(0, HALF)], sem).wait()
        pltpu.make_async_copy(table_hbm.at[idx_v.at[0].at[pl.ds(0, HALF)]],
                              buf.at[pl.ds(HALF, HALF)], sem).wait()

    gather(0, rows0)

    def body(q2, _):
        for k in range(2):
            q = 2 * q2 + k
            buf = bufs[k]

            @pl.when(q + 1 < Q_PER_W)
            def _():
                gather(q + 1, bufs[1 - k])

            drain(buf)
            pltpu.sync_copy(buf, out_hbm.at[pl.ds((q0 + q) * GROUP, GROUP)])
        return 0

    lax.fori_loop(0, Q_PER_W // 2, body, 0)


def _sc_gather(idx, table):
    mesh = plsc.VectorSubcoreMesh(core_axis_name="c", subcore_axis_name="s")
    k = functools.partial(
        pl.kernel,
        mesh=mesh,
        out_type=jax.ShapeDtypeStruct((NUM_ROWS, PAD_DIM), jnp.float32),
        scratch_types=[
            pltpu.VMEM((Q_PER_W, 2 * PAD_DIM), jnp.int32),
            pltpu.VMEM((GROUP, PAD_DIM), jnp.float32),
            pltpu.VMEM((GROUP, PAD_DIM), jnp.float32),
            pltpu.SemaphoreType.DMA,
        ],
    )(_gather_kernel)
    return k(idx, table)


def kernel(indices, weight):
    table = _tc_pad(weight.astype(jnp.float32))
    idx = jnp.pad(indices.reshape(2 * QUADS, HALF).astype(jnp.int32),
                  ((0, 0), (0, PAD_DIM - HALF))).reshape(QUADS, 2 * PAD_DIM)
    slab = _sc_gather(idx, table)
    return slab[:, :DIM].reshape(BATCH, SEQ, DIM)
